# baseline (speedup 1.0000x reference)
import functools

import jax
import jax.numpy as jnp
from jax import lax
from jax.experimental import pallas as pl
from jax.experimental.pallas import tpu as pltpu
from jax.experimental.pallas import tpu_sc as plsc

_N = 10000
_E = 320000
_D = 128
_NC = 2
_NS = 16
_NW = _NC * _NS
_EPW = _E // _NW
_K = 80
_NCH = _EPW // _K
_CH = 2000
_BR = 1280

_f32 = jnp.float32


def _mesh():
    return plsc.VectorSubcoreMesh(
        core_axis_name="c", subcore_axis_name="s",
        num_cores=_NC, num_subcores=_NS)


@functools.partial(
    pl.kernel,
    out_type=tuple(jax.ShapeDtypeStruct((_NW, _N), _f32) for _ in range(4)),
    mesh=_mesh(),
    compiler_params=pltpu.CompilerParams(needs_layout_passes=False),
    scratch_types=[
        pltpu.VMEM((_EPW,), jnp.int32),
        pltpu.VMEM((_N,), _f32),
    ],
)
def _deg_kernel(s0e, d0e, s1e, d1e, oAs, oAd, oBs, oBd, idx_v, hist):
    c = lax.axis_index("c")
    s = lax.axis_index("s")
    wid = c * _NS + s

    ones = jnp.ones((16,), _f32)

    for e_hbm, o_hbm in ((s0e, oAs), (d0e, oAd), (s1e, oBs), (d1e, oBd)):
        def zero_step(i, carry):
            hist[pl.ds(i * 16, 16)] = jnp.zeros((16,), _f32)
            return carry

        lax.fori_loop(0, _N // 16, zero_step, 0)
        pltpu.sync_copy(e_hbm.at[pl.ds(wid * _EPW, _EPW)], idx_v)

        def inner(j, carry):
            for u in range(4):
                idx = idx_v[pl.ds(j * 64 + u * 16, 16)]
                plsc.addupdate_scatter(hist, [idx], ones)
            return carry

        lax.fori_loop(0, _EPW // 64, inner, 0)
        pltpu.sync_copy(hist, o_hbm.at[wid])


_G = 1
_GE = _G * _K
_SEC = 5
_SCH = _NCH // _SEC
_SNG = _SCH // _G
_ZR = 80
_OSB = 2000


@functools.partial(
    pl.kernel,
    out_type=(jax.ShapeDtypeStruct((_NC, _N, _D), _f32),
              jax.ShapeDtypeStruct((_NC, _N, _D), _f32)),
    mesh=_mesh(),
    compiler_params=pltpu.CompilerParams(needs_layout_passes=False),
    scratch_types=[
        pltpu.VMEM((_SCH, _K), jnp.int32),
        pltpu.VMEM((_SCH, _K), jnp.int32),
        pltpu.VMEM((_GE, _D), _f32),
        pltpu.VMEM((_GE, _D), _f32),
        pltpu.VMEM_SHARED((_N, _D), _f32),
        pltpu.SemaphoreType.DMA,
        pltpu.SemaphoreType.DMA,
        pltpu.SemaphoreType.DMA,
    ],
)
def _agg_kernel(h0, s0e, d0e, h1, s1e, d1e, out0, out1,
                src2d, dst2d, bufA, bufB, agg, semA, semB, ssem):
    c = lax.axis_index("c")
    s = lax.axis_index("s")
    wid = c * _NS + s

    def zrow(i, carry):
        def zcol(k, carry2):
            bufA[i, pl.ds(k * 16, 16)] = jnp.zeros((16,), _f32)
            return carry2

        return lax.fori_loop(0, _D // 16, zcol, carry)

    def zero_agg():
        lax.fori_loop(0, _ZR, zrow, 0)

        @pl.when(s < _N // _OSB)
        def _():
            for j in range(_OSB // _ZR):
                pltpu.sync_copy(bufA.at[pl.ds(0, _ZR)],
                                agg.at[pl.ds(s * _OSB + j * _ZR, _ZR)])

    def run_layer(h_hbm, s_hbm, d_hbm, o_hbm):
        def issue_group(g, buf, sem):
            for j in range(_G):
                pltpu.async_copy(h_hbm.at[src2d.at[g * _G + j]],
                                 buf.at[pl.ds(j * _K, _K)], sem)

        def drain_group(buf, sem):
            pltpu.make_async_copy(h_hbm.at[pl.ds(0, _GE)], buf, sem).wait()

        def scatter_group(g, buf):
            descs = [
                pltpu.async_copy(buf.at[pl.ds(j * _K, _K)],
                                 agg.at[dst2d.at[g * _G + j]], ssem, add=True)
                for j in range(_G)
            ]
            for d in descs:
                d.wait()

        def section(sec, carry):
            pltpu.sync_copy(s_hbm.at[wid, sec], src2d)
            pltpu.sync_copy(d_hbm.at[wid, sec], dst2d)
            issue_group(0, bufA, semA)

            def pair(i, carry2):
                issue_group(2 * i + 1, bufB, semB)
                drain_group(bufA, semA)
                scatter_group(2 * i, bufA)
                issue_group(2 * i + 2, bufA, semA)
                drain_group(bufB, semB)
                scatter_group(2 * i + 1, bufB)
                return carry2

            lax.fori_loop(0, (_SNG - 1) // 2, pair, 0)
            drain_group(bufA, semA)
            scatter_group(_SNG - 1, bufA)
            return carry

        lax.fori_loop(0, _SEC, section, 0)

        plsc.subcore_barrier()

        @pl.when(s < _N // _OSB)
        def _():
            pltpu.sync_copy(agg.at[pl.ds(s * _OSB, _OSB)],
                            o_hbm.at[c, pl.ds(s * _OSB, _OSB)])

    zero_agg()
    plsc.subcore_barrier()
    run_layer(h0, s0e, d0e, out0)
    zero_agg()
    plsc.subcore_barrier()
    run_layer(h1, s1e, d1e, out1)


def _prep_body(x0_ref, x1_ref, dsa_ref, dsb_ref, h0_ref, h1_ref):
    s0 = lax.rsqrt(jnp.maximum(jnp.sum(dsa_ref[...], axis=0), 1.0))
    s1 = lax.rsqrt(jnp.maximum(jnp.sum(dsb_ref[...], axis=0), 1.0))
    h0_ref[...] = x0_ref[...] * s0[:, None]
    h1_ref[...] = x1_ref[...] * s1[:, None]


def _final_body(p0_ref, p1_ref, dda_ref, ddb_ref, w0_ref, b0_ref,
                w1_ref, b1_ref, o0_ref, o1_ref):
    n0 = lax.rsqrt(jnp.maximum(jnp.sum(dda_ref[...], axis=0), 1.0))[:, None]
    n1 = lax.rsqrt(jnp.maximum(jnp.sum(ddb_ref[...], axis=0), 1.0))[:, None]
    a0 = (p0_ref[0] + p0_ref[1]) * n0
    a1 = (p1_ref[0] + p1_ref[1]) * n1
    o0_ref[...] = jnp.maximum(
        jnp.dot(a0, w0_ref[...], preferred_element_type=_f32) + b0_ref[...],
        0.0)
    o1_ref[...] = jnp.maximum(
        jnp.dot(a1, w1_ref[...], preferred_element_type=_f32) + b1_ref[...],
        0.0)


def _prep_call(x0, x1, dsa, dsb):
    grid = (pl.cdiv(_N, _BR),)
    xspec = pl.BlockSpec((_BR, _D), lambda i: (i, 0))
    dspec = pl.BlockSpec((_NW, _BR), lambda i: (0, i))
    return pl.pallas_call(
        _prep_body,
        grid=grid,
        in_specs=[xspec, xspec, dspec, dspec],
        out_specs=[xspec, xspec],
        out_shape=[jax.ShapeDtypeStruct((_N, _D), _f32)] * 2,
    )(x0, x1, dsa, dsb)


def _final_call(p0, p1, dda, ddb, w0, b0, w1, b1):
    grid = (pl.cdiv(_N, _BR),)
    pspec = pl.BlockSpec((_NC, _BR, _D), lambda i: (0, i, 0))
    dspec = pl.BlockSpec((_NW, _BR), lambda i: (0, i))
    wspec = pl.BlockSpec((_D, _D), lambda i: (0, 0))
    bspec = pl.BlockSpec((1, _D), lambda i: (0, 0))
    ospec = pl.BlockSpec((_BR, _D), lambda i: (i, 0))
    return pl.pallas_call(
        _final_body,
        grid=grid,
        in_specs=[pspec, pspec, dspec, dspec, wspec, bspec, wspec, bspec],
        out_specs=[ospec, ospec],
        out_shape=[jax.ShapeDtypeStruct((_N, _D), _f32)] * 2,
    )(p0, p1, dda, ddb, w0, b0, w1, b1)


def kernel(x0, edge_index0, x1, edge_index1, W0, b0, W1, b1):
    s0e, d0e = edge_index0[0], edge_index0[1]
    s1e, d1e = edge_index1[0], edge_index1[1]
    dAs, dAd, dBs, dBd = _deg_kernel(s0e, d0e, s1e, d1e)
    h0, h1 = _prep_call(x0, x1, dAs, dBs)
    esh = (_NW, _SEC, _SCH, _K)
    p0, p1 = _agg_kernel(
        h0, s0e.reshape(esh), d0e.reshape(esh),
        h1, s1e.reshape(esh), d1e.reshape(esh))
    out0, out1 = _final_call(p0, p1, dAd, dBd,
                             W0, b0.reshape(1, _D), W1, b1.reshape(1, _D))
    return out0, out1


# speedup vs baseline: 1.0360x; 1.0360x over previous
import functools

import jax
import jax.numpy as jnp
from jax import lax
from jax.experimental import pallas as pl
from jax.experimental.pallas import tpu as pltpu
from jax.experimental.pallas import tpu_sc as plsc

_N = 10000
_E = 320000
_D = 128
_NC = 2
_NS = 16
_NW = _NC * _NS
_EPW = _E // _NW
_K = 128
_TK = 16
_NCHM = 78
_CH = 2000
_BR = 1280

_f32 = jnp.float32


def _mesh():
    return plsc.VectorSubcoreMesh(
        core_axis_name="c", subcore_axis_name="s",
        num_cores=_NC, num_subcores=_NS)


@functools.partial(
    pl.kernel,
    out_type=tuple(jax.ShapeDtypeStruct((_NW, _N), _f32) for _ in range(4)),
    mesh=_mesh(),
    compiler_params=pltpu.CompilerParams(needs_layout_passes=False),
    scratch_types=[
        pltpu.VMEM((_EPW,), jnp.int32),
        pltpu.VMEM((_N,), _f32),
    ],
)
def _deg_kernel(s0e, d0e, s1e, d1e, oAs, oAd, oBs, oBd, idx_v, hist):
    c = lax.axis_index("c")
    s = lax.axis_index("s")
    wid = c * _NS + s

    ones = jnp.ones((16,), _f32)

    for e_hbm, o_hbm in ((s0e, oAs), (d0e, oAd), (s1e, oBs), (d1e, oBd)):
        def zero_step(i, carry):
            hist[pl.ds(i * 16, 16)] = jnp.zeros((16,), _f32)
            return carry

        lax.fori_loop(0, _N // 16, zero_step, 0)
        pltpu.sync_copy(e_hbm.at[pl.ds(wid * _EPW, _EPW)], idx_v)

        def inner(j, carry):
            idx = idx_v[pl.ds(j * 16, 16)]
            plsc.addupdate_scatter(hist, [idx], ones)
            return carry

        lax.fori_loop(0, _EPW // 16, inner, 0)
        pltpu.sync_copy(hist, o_hbm.at[wid])


_G = 1
_GE = _G * _K
_SEC = 6
_SCH = _NCHM // _SEC
_SNG = _SCH // _G
_ZR = 80
_OSB = 2000


@functools.partial(
    pl.kernel,
    out_type=(jax.ShapeDtypeStruct((_NC, _N, _D), _f32),
              jax.ShapeDtypeStruct((_NC, _N, _D), _f32)),
    mesh=_mesh(),
    compiler_params=pltpu.CompilerParams(needs_layout_passes=False),
    scratch_types=[
        pltpu.VMEM((_SCH, _K), jnp.int32),
        pltpu.VMEM((_SCH, _K), jnp.int32),
        pltpu.VMEM((_TK,), jnp.int32),
        pltpu.VMEM((1, _TK), jnp.int32),
        pltpu.VMEM((_GE, _D), _f32),
        pltpu.VMEM((_GE, _D), _f32),
        pltpu.VMEM_SHARED((_N, _D), _f32),
        pltpu.SemaphoreType.DMA,
        pltpu.SemaphoreType.DMA,
        pltpu.SemaphoreType.DMA,
    ],
)
def _agg_kernel(h0, s0e, d0e, s0t, d0t, h1, s1e, d1e, s1t, d1t, out0, out1,
                src2d, dst2d, st_v, dt_v, bufA, bufB, agg, semA, semB, ssem):
    c = lax.axis_index("c")
    s = lax.axis_index("s")
    wid = c * _NS + s

    def zrow(i, carry):
        def zcol(k, carry2):
            bufA[i, pl.ds(k * 16, 16)] = jnp.zeros((16,), _f32)
            return carry2

        return lax.fori_loop(0, _D // 16, zcol, carry)

    def zero_agg():
        lax.fori_loop(0, _ZR, zrow, 0)

        @pl.when(s < _N // _OSB)
        def _():
            for j in range(_OSB // _ZR):
                pltpu.sync_copy(bufA.at[pl.ds(0, _ZR)],
                                agg.at[pl.ds(s * _OSB + j * _ZR, _ZR)])

    def run_layer(h_hbm, s_hbm, d_hbm, st_hbm, dt_hbm, o_hbm):
        def issue_group(g, buf, sem):
            for j in range(_G):
                pltpu.async_copy(h_hbm.at[src2d.at[g * _G + j]],
                                 buf.at[pl.ds(j * _K, _K)], sem)

        def drain_group(buf, sem):
            pltpu.make_async_copy(h_hbm.at[pl.ds(0, _GE)], buf, sem).wait()

        def scatter_group(g, buf):
            descs = [
                pltpu.async_copy(buf.at[pl.ds(j * _K, _K)],
                                 agg.at[dst2d.at[g * _G + j]], ssem, add=True)
                for j in range(_G)
            ]
            for d in descs:
                d.wait()

        def section(sec, carry):
            pltpu.sync_copy(s_hbm.at[wid, sec], src2d)
            pltpu.sync_copy(d_hbm.at[wid, sec], dst2d)
            issue_group(0, bufA, semA)

            def pair(i, carry2):
                issue_group(2 * i + 1, bufB, semB)
                drain_group(bufA, semA)
                scatter_group(2 * i, bufA)
                issue_group(2 * i + 2, bufA, semA)
                drain_group(bufB, semB)
                scatter_group(2 * i + 1, bufB)
                return carry2

            lax.fori_loop(0, (_SNG - 1) // 2, pair, 0)
            drain_group(bufA, semA)
            scatter_group(_SNG - 1, bufA)
            return carry

        lax.fori_loop(0, _SEC, section, 0)

        pltpu.sync_copy(st_hbm.at[pl.ds(wid * _TK, _TK)], st_v)
        pltpu.sync_copy(dt_hbm.at[wid], dt_v)
        pltpu.async_copy(h_hbm.at[st_v], bufA.at[pl.ds(0, _TK)], semA).wait()
        pltpu.sync_copy(bufA.at[pl.ds(0, _TK)], agg.at[dt_v.at[0]], add=True)

        plsc.subcore_barrier()

        @pl.when(s < _N // _OSB)
        def _():
            pltpu.sync_copy(agg.at[pl.ds(s * _OSB, _OSB)],
                            o_hbm.at[c, pl.ds(s * _OSB, _OSB)])

    zero_agg()
    plsc.subcore_barrier()
    run_layer(h0, s0e, d0e, s0t, d0t, out0)
    zero_agg()
    plsc.subcore_barrier()
    run_layer(h1, s1e, d1e, s1t, d1t, out1)


def _prep_body(x0_ref, x1_ref, dsa_ref, dsb_ref, h0_ref, h1_ref):
    s0 = lax.rsqrt(jnp.maximum(jnp.sum(dsa_ref[...], axis=0), 1.0))
    s1 = lax.rsqrt(jnp.maximum(jnp.sum(dsb_ref[...], axis=0), 1.0))
    h0_ref[...] = x0_ref[...] * s0[:, None]
    h1_ref[...] = x1_ref[...] * s1[:, None]


def _final_body(p0_ref, p1_ref, dda_ref, ddb_ref, w0_ref, b0_ref,
                w1_ref, b1_ref, o0_ref, o1_ref):
    n0 = lax.rsqrt(jnp.maximum(jnp.sum(dda_ref[...], axis=0), 1.0))[:, None]
    n1 = lax.rsqrt(jnp.maximum(jnp.sum(ddb_ref[...], axis=0), 1.0))[:, None]
    a0 = (p0_ref[0] + p0_ref[1]) * n0
    a1 = (p1_ref[0] + p1_ref[1]) * n1
    o0_ref[...] = jnp.maximum(
        jnp.dot(a0, w0_ref[...], preferred_element_type=_f32) + b0_ref[...],
        0.0)
    o1_ref[...] = jnp.maximum(
        jnp.dot(a1, w1_ref[...], preferred_element_type=_f32) + b1_ref[...],
        0.0)


def _prep_call(x0, x1, dsa, dsb):
    grid = (pl.cdiv(_N, _BR),)
    xspec = pl.BlockSpec((_BR, _D), lambda i: (i, 0))
    dspec = pl.BlockSpec((_NW, _BR), lambda i: (0, i))
    return pl.pallas_call(
        _prep_body,
        grid=grid,
        in_specs=[xspec, xspec, dspec, dspec],
        out_specs=[xspec, xspec],
        out_shape=[jax.ShapeDtypeStruct((_N, _D), _f32)] * 2,
    )(x0, x1, dsa, dsb)


def _final_call(p0, p1, dda, ddb, w0, b0, w1, b1):
    grid = (pl.cdiv(_N, _BR),)
    pspec = pl.BlockSpec((_NC, _BR, _D), lambda i: (0, i, 0))
    dspec = pl.BlockSpec((_NW, _BR), lambda i: (0, i))
    wspec = pl.BlockSpec((_D, _D), lambda i: (0, 0))
    bspec = pl.BlockSpec((1, _D), lambda i: (0, 0))
    ospec = pl.BlockSpec((_BR, _D), lambda i: (i, 0))
    return pl.pallas_call(
        _final_body,
        grid=grid,
        in_specs=[pspec, pspec, dspec, dspec, wspec, bspec, wspec, bspec],
        out_specs=[ospec, ospec],
        out_shape=[jax.ShapeDtypeStruct((_N, _D), _f32)] * 2,
    )(p0, p1, dda, ddb, w0, b0, w1, b1)


def kernel(x0, edge_index0, x1, edge_index1, W0, b0, W1, b1):
    s0e, d0e = edge_index0[0], edge_index0[1]
    s1e, d1e = edge_index1[0], edge_index1[1]
    dAs, dAd, dBs, dBd = _deg_kernel(s0e, d0e, s1e, d1e)
    h0, h1 = _prep_call(x0, x1, dAs, dBs)
    esh = (_NW, _SEC, _SCH, _K)

    def _split(e):
        e2 = e.reshape(_NW, _EPW)
        main = e2[:, :_NCHM * _K].reshape(esh)
        return main, e2[:, _NCHM * _K:]

    s0m, s0t = _split(s0e)
    d0m, d0t = _split(d0e)
    s1m, s1t = _split(s1e)
    d1m, d1t = _split(d1e)
    p0, p1 = _agg_kernel(
        h0, s0m, d0m, s0t.reshape(-1), d0t.reshape(_NW, 1, _TK),
        h1, s1m, d1m, s1t.reshape(-1), d1t.reshape(_NW, 1, _TK))
    out0, out1 = _final_call(p0, p1, dAd, dBd,
                             W0, b0.reshape(1, _D), W1, b1.reshape(1, _D))
    return out0, out1


# speedup vs baseline: 1.1539x; 1.1139x over previous
import functools

import jax
import jax.numpy as jnp
from jax import lax
from jax.experimental import pallas as pl
from jax.experimental.pallas import tpu as pltpu
from jax.experimental.pallas import tpu_sc as plsc

_N = 10000
_E = 320000
_D = 128
_NC = 2
_NS = 16
_NW = _NC * _NS
_EPW = _E // _NW
_K = 128
_TK = 16
_NCHM = 78
_CH = 2000
_BR = 1280

_f32 = jnp.float32


def _mesh():
    return plsc.VectorSubcoreMesh(
        core_axis_name="c", subcore_axis_name="s",
        num_cores=_NC, num_subcores=_NS)


@functools.partial(
    pl.kernel,
    out_type=tuple(jax.ShapeDtypeStruct((_NW, _N), _f32) for _ in range(4)),
    mesh=_mesh(),
    compiler_params=pltpu.CompilerParams(needs_layout_passes=False),
    scratch_types=[
        pltpu.VMEM((_EPW,), jnp.int32),
        pltpu.VMEM((_EPW,), jnp.int32),
        pltpu.VMEM((_N,), _f32),
        pltpu.SemaphoreType.DMA,
    ],
)
def _deg_kernel(s0e, d0e, s1e, d1e, oAs, oAd, oBs, oBd,
                idx_v, idx_w, hist, dsem):
    c = lax.axis_index("c")
    s = lax.axis_index("s")
    wid = c * _NS + s

    ones = jnp.ones((16,), _f32)

    arrs = ((s0e, oAs), (d0e, oAd), (s1e, oBs), (d1e, oBd))
    pltpu.sync_copy(arrs[0][0].at[pl.ds(wid * _EPW, _EPW)], idx_v)
    for a, (e_hbm, o_hbm) in enumerate(arrs):
        buf = idx_v if a % 2 == 0 else idx_w
        nxt = idx_w if a % 2 == 0 else idx_v
        desc = None
        if a + 1 < len(arrs):
            desc = pltpu.async_copy(
                arrs[a + 1][0].at[pl.ds(wid * _EPW, _EPW)], nxt, dsem)

        def zero_step(i, carry):
            z = jnp.zeros((16,), _f32)
            for u in range(4):
                hist[pl.ds(i * 64 + u * 16, 16)] = z
            return carry

        lax.fori_loop(0, _N // 64, zero_step, 0)

        def inner(j, carry, buf=buf):
            idx = buf[pl.ds(j * 16, 16)]
            plsc.addupdate_scatter(hist, [idx], ones)
            return carry

        lax.fori_loop(0, _EPW // 16, inner, 0)
        pltpu.sync_copy(hist, o_hbm.at[wid])
        if desc is not None:
            desc.wait()


_G = 1
_GE = _G * _K
_SEC = 2
_SCH = _NCHM // _SEC
_SNG = _SCH // _G
_ZR = 80
_OSB = 2000


@functools.partial(
    pl.kernel,
    out_type=(jax.ShapeDtypeStruct((_NC, _N, _D), _f32),
              jax.ShapeDtypeStruct((_NC, _N, _D), _f32)),
    mesh=_mesh(),
    compiler_params=pltpu.CompilerParams(needs_layout_passes=False),
    scratch_types=[
        pltpu.VMEM((_SCH, _K), jnp.int32),
        pltpu.VMEM((_SCH, _K), jnp.int32),
        pltpu.VMEM((_TK,), jnp.int32),
        pltpu.VMEM((1, _TK), jnp.int32),
        pltpu.VMEM((_GE, _D), _f32),
        pltpu.VMEM((_GE, _D), _f32),
        pltpu.VMEM_SHARED((_N, _D), _f32),
        pltpu.SemaphoreType.DMA,
        pltpu.SemaphoreType.DMA,
        pltpu.SemaphoreType.DMA,
    ],
)
def _agg_kernel(h0, s0e, d0e, s0t, d0t, h1, s1e, d1e, s1t, d1t, out0, out1,
                src2d, dst2d, st_v, dt_v, bufA, bufB, agg, semA, semB, ssem):
    c = lax.axis_index("c")
    s = lax.axis_index("s")
    wid = c * _NS + s

    def zrow(i, carry):
        def zcol(k, carry2):
            bufA[i, pl.ds(k * 16, 16)] = jnp.zeros((16,), _f32)
            return carry2

        return lax.fori_loop(0, _D // 16, zcol, carry)

    def zero_agg():
        lax.fori_loop(0, _ZR, zrow, 0)

        @pl.when(s < _N // _OSB)
        def _():
            for j in range(_OSB // _ZR):
                pltpu.sync_copy(bufA.at[pl.ds(0, _ZR)],
                                agg.at[pl.ds(s * _OSB + j * _ZR, _ZR)])

    def run_layer(h_hbm, s_hbm, d_hbm, st_hbm, dt_hbm, o_hbm):
        def issue_group(g, buf, sem):
            for j in range(_G):
                pltpu.async_copy(h_hbm.at[src2d.at[g * _G + j]],
                                 buf.at[pl.ds(j * _K, _K)], sem)

        def drain_group(buf, sem):
            pltpu.make_async_copy(h_hbm.at[pl.ds(0, _GE)], buf, sem).wait()

        def scatter_group(g, buf):
            descs = [
                pltpu.async_copy(buf.at[pl.ds(j * _K, _K)],
                                 agg.at[dst2d.at[g * _G + j]], ssem, add=True)
                for j in range(_G)
            ]
            for d in descs:
                d.wait()

        def section(sec, carry):
            pltpu.sync_copy(s_hbm.at[wid, sec], src2d)
            pltpu.sync_copy(d_hbm.at[wid, sec], dst2d)
            issue_group(0, bufA, semA)

            def pair(i, carry2):
                issue_group(2 * i + 1, bufB, semB)
                drain_group(bufA, semA)
                scatter_group(2 * i, bufA)
                issue_group(2 * i + 2, bufA, semA)
                drain_group(bufB, semB)
                scatter_group(2 * i + 1, bufB)
                return carry2

            lax.fori_loop(0, (_SNG - 1) // 2, pair, 0)
            drain_group(bufA, semA)
            scatter_group(_SNG - 1, bufA)
            return carry

        lax.fori_loop(0, _SEC, section, 0)

        pltpu.sync_copy(st_hbm.at[pl.ds(wid * _TK, _TK)], st_v)
        pltpu.sync_copy(dt_hbm.at[wid], dt_v)
        pltpu.async_copy(h_hbm.at[st_v], bufA.at[pl.ds(0, _TK)], semA).wait()
        pltpu.sync_copy(bufA.at[pl.ds(0, _TK)], agg.at[dt_v.at[0]], add=True)

        plsc.subcore_barrier()

        @pl.when(s < _N // _OSB)
        def _():
            pltpu.sync_copy(agg.at[pl.ds(s * _OSB, _OSB)],
                            o_hbm.at[c, pl.ds(s * _OSB, _OSB)])

    zero_agg()
    plsc.subcore_barrier()
    run_layer(h0, s0e, d0e, s0t, d0t, out0)
    zero_agg()
    plsc.subcore_barrier()
    run_layer(h1, s1e, d1e, s1t, d1t, out1)


def _prep_body(x0_ref, x1_ref, dsa_ref, dsb_ref, h0_ref, h1_ref):
    s0 = lax.rsqrt(jnp.maximum(jnp.sum(dsa_ref[...], axis=0), 1.0))
    s1 = lax.rsqrt(jnp.maximum(jnp.sum(dsb_ref[...], axis=0), 1.0))
    h0_ref[...] = x0_ref[...] * s0[:, None]
    h1_ref[...] = x1_ref[...] * s1[:, None]


def _final_body(p0_ref, p1_ref, dda_ref, ddb_ref, w0_ref, b0_ref,
                w1_ref, b1_ref, o0_ref, o1_ref):
    n0 = lax.rsqrt(jnp.maximum(jnp.sum(dda_ref[...], axis=0), 1.0))[:, None]
    n1 = lax.rsqrt(jnp.maximum(jnp.sum(ddb_ref[...], axis=0), 1.0))[:, None]
    a0 = (p0_ref[0] + p0_ref[1]) * n0
    a1 = (p1_ref[0] + p1_ref[1]) * n1
    o0_ref[...] = jnp.maximum(
        jnp.dot(a0, w0_ref[...], preferred_element_type=_f32) + b0_ref[...],
        0.0)
    o1_ref[...] = jnp.maximum(
        jnp.dot(a1, w1_ref[...], preferred_element_type=_f32) + b1_ref[...],
        0.0)


def _prep_call(x0, x1, dsa, dsb):
    grid = (pl.cdiv(_N, _BR),)
    xspec = pl.BlockSpec((_BR, _D), lambda i: (i, 0))
    dspec = pl.BlockSpec((_NW, _BR), lambda i: (0, i))
    return pl.pallas_call(
        _prep_body,
        grid=grid,
        in_specs=[xspec, xspec, dspec, dspec],
        out_specs=[xspec, xspec],
        out_shape=[jax.ShapeDtypeStruct((_N, _D), _f32)] * 2,
    )(x0, x1, dsa, dsb)


def _final_call(p0, p1, dda, ddb, w0, b0, w1, b1):
    grid = (pl.cdiv(_N, _BR),)
    pspec = pl.BlockSpec((_NC, _BR, _D), lambda i: (0, i, 0))
    dspec = pl.BlockSpec((_NW, _BR), lambda i: (0, i))
    wspec = pl.BlockSpec((_D, _D), lambda i: (0, 0))
    bspec = pl.BlockSpec((1, _D), lambda i: (0, 0))
    ospec = pl.BlockSpec((_BR, _D), lambda i: (i, 0))
    return pl.pallas_call(
        _final_body,
        grid=grid,
        in_specs=[pspec, pspec, dspec, dspec, wspec, bspec, wspec, bspec],
        out_specs=[ospec, ospec],
        out_shape=[jax.ShapeDtypeStruct((_N, _D), _f32)] * 2,
    )(p0, p1, dda, ddb, w0, b0, w1, b1)


def kernel(x0, edge_index0, x1, edge_index1, W0, b0, W1, b1):
    s0e, d0e = edge_index0[0], edge_index0[1]
    s1e, d1e = edge_index1[0], edge_index1[1]
    dAs, dAd, dBs, dBd = _deg_kernel(s0e, d0e, s1e, d1e)
    h0, h1 = _prep_call(x0, x1, dAs, dBs)
    esh = (_NW, _SEC, _SCH, _K)

    def _split(e):
        e2 = e.reshape(_NW, _EPW)
        main = e2[:, :_NCHM * _K].reshape(esh)
        return main, e2[:, _NCHM * _K:]

    s0m, s0t = _split(s0e)
    d0m, d0t = _split(d0e)
    s1m, s1t = _split(s1e)
    d1m, d1t = _split(d1e)
    p0, p1 = _agg_kernel(
        h0, s0m, d0m, s0t.reshape(-1), d0t.reshape(_NW, 1, _TK),
        h1, s1m, d1m, s1t.reshape(-1), d1t.reshape(_NW, 1, _TK))
    out0, out1 = _final_call(p0, p1, dAd, dBd,
                             W0, b0.reshape(1, _D), W1, b1.reshape(1, _D))
    return out0, out1
